# Initial kernel scaffold; baseline (speedup 1.0000x reference)
import jax
import jax.numpy as jnp
from jax.experimental import pallas as pl


def kernel(p, boundaries, planes_xy, planes_xz, planes_yz, c_planes_xy, c_planes_xz, c_planes_yz, W0, b0, W1, b1, Wout, bout, cW0, cb0, cW1, cb1, cWout, cbout):
    raise NotImplementedError("write your pallas kernel here")


# speedup vs baseline: 168.8305x; 168.8305x over previous
import functools

import jax
import jax.numpy as jnp
from jax import lax
from jax.experimental import pallas as pl
from jax.experimental.pallas import tpu as pltpu
from jax.experimental.pallas import tpu_sc as plsc

S = 8
R = 128
IN_DIM = 32
HID = 32
N = 262144

NC = 2
NS = 16
NW = NC * NS
PPW = N // NW
CH = 64
NCHUNK = PPW // CH


def _prep_body(b_ref, px_ref, py_ref, pz_ref, idx_ref, w_ref):
    px = px_ref[...]
    py = py_ref[...]
    pz = pz_ref[...]
    shp = px.shape
    pre = jnp.zeros(shp, jnp.bool_)
    lox = jnp.zeros(shp, jnp.float32)
    loy = jnp.zeros(shp, jnp.float32)
    loz = jnp.zeros(shp, jnp.float32)
    hix = jnp.ones(shp, jnp.float32)
    hiy = jnp.ones(shp, jnp.float32)
    hiz = jnp.ones(shp, jnp.float32)
    sidx = jnp.zeros(shp, jnp.int32)
    for s in range(S):
        l0 = b_ref[s, 0, 0]
        l1 = b_ref[s, 0, 1]
        l2 = b_ref[s, 0, 2]
        h0 = b_ref[s, 1, 0]
        h1 = b_ref[s, 1, 1]
        h2 = b_ref[s, 1, 2]
        m = ((px > l0) & (px < h0) & (py > l1) & (py < h1)
             & (pz > l2) & (pz < h2) & (~pre))
        pre = pre | m
        lox = jnp.where(m, l0, lox)
        loy = jnp.where(m, l1, loy)
        loz = jnp.where(m, l2, loz)
        hix = jnp.where(m, h0, hix)
        hiy = jnp.where(m, h1, hiy)
        hiz = jnp.where(m, h2, hiz)
        sidx = jnp.where(m, s, sidx)
    routed = pre.astype(jnp.float32)
    dx = jnp.where(pre, hix - lox, 1.0)
    dy = jnp.where(pre, hiy - loy, 1.0)
    dz = jnp.where(pre, hiz - loz, 1.0)
    un = (px - lox) / dx
    vn = (py - loy) / dy
    tn = (pz - loz) / dz
    sbase = sidx * (R * R)
    for o, (ca, cb) in enumerate(((un, vn), (un, tn), (vn, tn))):
        xx = jnp.clip(ca, 0.0, 1.0) * (R - 1)
        yy = jnp.clip(cb, 0.0, 1.0) * (R - 1)
        x0 = jnp.clip(jnp.floor(xx), 0, R - 2).astype(jnp.int32)
        y0 = jnp.clip(jnp.floor(yy), 0, R - 2).astype(jnp.int32)
        wx = xx - x0.astype(jnp.float32)
        wy = yy - y0.astype(jnp.float32)
        base = sbase + x0 * R + y0
        idx_ref[4 * o + 0] = base
        idx_ref[4 * o + 1] = base + 1
        idx_ref[4 * o + 2] = base + R
        idx_ref[4 * o + 3] = base + R + 1
        w_ref[4 * o + 0] = (1 - wx) * (1 - wy) * routed
        w_ref[4 * o + 1] = (1 - wx) * wy * routed
        w_ref[4 * o + 2] = wx * (1 - wy) * routed
        w_ref[4 * o + 3] = wx * wy * routed


def _prep(px, py, pz, boundaries):
    nb = px.shape[0]
    blk = 256
    grid = nb // blk
    return pl.pallas_call(
        _prep_body,
        grid=(grid,),
        in_specs=[
            pl.BlockSpec(memory_space=pltpu.SMEM),
            pl.BlockSpec((blk, 128), lambda i: (i, 0)),
            pl.BlockSpec((blk, 128), lambda i: (i, 0)),
            pl.BlockSpec((blk, 128), lambda i: (i, 0)),
        ],
        out_specs=[
            pl.BlockSpec((12, blk, 128), lambda i: (0, i, 0)),
            pl.BlockSpec((12, blk, 128), lambda i: (0, i, 0)),
        ],
        out_shape=[
            jax.ShapeDtypeStruct((12, nb, 128), jnp.int32),
            jax.ShapeDtypeStruct((12, nb, 128), jnp.float32),
        ],
    )(boundaries, px, py, pz)


def _sc_body(idx_hbm, w_hbm, txy, txz, tyz, ctxy, ctxz, ctyz,
             feat_hbm, cfeat_hbm, *scr):
    idx_v = scr[0]
    w_v = scr[1]
    rows = scr[2:14]
    crows = scr[14:26]
    featb = scr[26]
    cfeatb = scr[27]
    sem = scr[28]
    tabs = (txy, txz, tyz)
    ctabs = (ctxy, ctxz, ctyz)

    wid = lax.axis_index("s") * NC + lax.axis_index("c")
    base0 = wid * PPW
    iota16 = lax.iota(jnp.int32, 16)

    def chunk_body(i, carry):
        base = pl.multiple_of(base0 + i * CH, CH)
        pltpu.sync_copy(idx_hbm.at[:, pl.ds(base, CH)], idx_v)
        pltpu.sync_copy(w_hbm.at[:, pl.ds(base, CH)], w_v)
        cps = []
        for j in range(12):
            o = j // 4
            cps.append(pltpu.async_copy(tabs[o].at[idx_v.at[j]], rows[j], sem))
            cps.append(pltpu.async_copy(ctabs[o].at[idx_v.at[j]], crows[j], sem))
        for cp in cps:
            cp.wait()
        for g in range(CH // 16):
            ptv = g * 16 + iota16
            wvs = [w_v[j, pl.ds(g * 16, 16)] for j in range(12)]

            def c_body(c, carry2):
                cs = jnp.full((16,), 0, jnp.int32) + c
                facc = wvs[0] * plsc.load_gather(rows[0], [ptv, cs])
                cacc = wvs[0] * plsc.load_gather(crows[0], [ptv, cs])
                for j in range(1, 12):
                    facc = facc + wvs[j] * plsc.load_gather(rows[j], [ptv, cs])
                    cacc = cacc + wvs[j] * plsc.load_gather(crows[j], [ptv, cs])
                plsc.store_scatter(featb, [ptv, cs], facc)
                plsc.store_scatter(cfeatb, [ptv, cs], cacc)
                return carry2

            lax.fori_loop(0, IN_DIM, c_body, 0)
        pltpu.sync_copy(featb, feat_hbm.at[pl.ds(base, CH)])
        pltpu.sync_copy(cfeatb, cfeat_hbm.at[pl.ds(base, CH)])
        return carry

    lax.fori_loop(0, NCHUNK, chunk_body, 0)


def _gather_sc(idx, w, txy, txz, tyz, ctxy, ctxz, ctyz):
    mesh = plsc.VectorSubcoreMesh(
        core_axis_name="c", subcore_axis_name="s",
        num_cores=NC, num_subcores=NS)
    scratch = (
        [pltpu.VMEM((12, CH), jnp.int32), pltpu.VMEM((12, CH), jnp.float32)]
        + [pltpu.VMEM((CH, IN_DIM), jnp.float32) for _ in range(24)]
        + [pltpu.VMEM((CH, IN_DIM), jnp.float32) for _ in range(2)]
        + [pltpu.SemaphoreType.DMA]
    )
    fn = pl.kernel(
        _sc_body,
        out_type=[
            jax.ShapeDtypeStruct((N, IN_DIM), jnp.float32),
            jax.ShapeDtypeStruct((N, IN_DIM), jnp.float32),
        ],
        mesh=mesh,
        scratch_types=scratch,
        compiler_params=pltpu.CompilerParams(use_tc_tiling_on_sc=False,
                                             needs_layout_passes=False),
    )
    return fn(idx, w, txy, txz, tyz, ctxy, ctxz, ctyz)


def _mlp_body(f_ref, c_ref, W0r, b0r, W1r, b1r, Wor, bor,
              cW0r, cb0r, cW1r, cb1r, cWor, cbor, out_ref):
    f = f_ref[...]
    c = c_ref[...]
    h = jnp.maximum(jnp.dot(f, W0r[...], preferred_element_type=jnp.float32)
                    + b0r[...], 0.0)
    h = jnp.maximum(jnp.dot(h, W1r[...], preferred_element_type=jnp.float32)
                    + b1r[...], 0.0)
    sdf = jnp.tanh(jnp.dot(h, Wor[...], preferred_element_type=jnp.float32)
                   + bor[...])
    hc = jnp.maximum(jnp.dot(c, cW0r[...], preferred_element_type=jnp.float32)
                     + cb0r[...], 0.0)
    hc = jnp.maximum(jnp.dot(hc, cW1r[...], preferred_element_type=jnp.float32)
                     + cb1r[...], 0.0)
    rgb = jax.nn.sigmoid(
        jnp.dot(hc, cWor[...], preferred_element_type=jnp.float32) + cbor[...])
    out_ref[...] = jnp.concatenate([rgb, sdf], axis=1)


def _mlp(feat, cfeat, W0, b0, W1, b1, Wout, bout, cW0, cb0, cW1, cb1,
         cWout, cbout):
    blk = 2048
    grid = N // blk

    def fullspec(a):
        return pl.BlockSpec(a.shape, lambda i: (0,) * a.ndim)

    ws = [W0, b0.reshape(1, HID), W1, b1.reshape(1, HID),
          Wout, bout.reshape(1, 1), cW0, cb0.reshape(1, HID),
          cW1, cb1.reshape(1, HID), cWout, cbout.reshape(1, 3)]
    return pl.pallas_call(
        _mlp_body,
        grid=(grid,),
        in_specs=([pl.BlockSpec((blk, IN_DIM), lambda i: (i, 0)),
                   pl.BlockSpec((blk, IN_DIM), lambda i: (i, 0))]
                  + [fullspec(a) for a in ws]),
        out_specs=pl.BlockSpec((blk, 4), lambda i: (i, 0)),
        out_shape=jax.ShapeDtypeStruct((N, 4), jnp.float32),
    )(feat, cfeat, *ws)


def kernel(p, boundaries, planes_xy, planes_xz, planes_yz,
           c_planes_xy, c_planes_xz, c_planes_yz,
           W0, b0, W1, b1, Wout, bout, cW0, cb0, cW1, cb1, cWout, cbout):
    nb = N // 128
    px = p[:, 0].reshape(nb, 128)
    py = p[:, 1].reshape(nb, 128)
    pz = p[:, 2].reshape(nb, 128)
    idx3, w3 = _prep(px, py, pz, boundaries)
    idx = idx3.reshape(12, N)
    w = w3.reshape(12, N)
    tabs = [a.reshape(S * R * R, IN_DIM)
            for a in (planes_xy, planes_xz, planes_yz,
                      c_planes_xy, c_planes_xz, c_planes_yz)]
    feat, cfeat = _gather_sc(idx, w, *tabs)
    return _mlp(feat, cfeat, W0, b0, W1, b1, Wout, bout,
                cW0, cb0, cW1, cb1, cWout, cbout)


# speedup vs baseline: 180.7428x; 1.0706x over previous
import jax
import jax.numpy as jnp
from jax import lax
from jax.experimental import pallas as pl
from jax.experimental.pallas import tpu as pltpu
from jax.experimental.pallas import tpu_sc as plsc

S = 8
R = 128
IN_DIM = 32
HID = 32
N = 262144
D2 = 2 * IN_DIM

NC = 2
NS = 16
NW = NC * NS
PPW = N // NW
CH = 64
NCHUNK = PPW // CH


def _prep_body(b_ref, px_ref, py_ref, pz_ref, iw_ref):
    px = px_ref[...]
    py = py_ref[...]
    pz = pz_ref[...]
    shp = px.shape
    pre = jnp.zeros(shp, jnp.bool_)
    lox = jnp.zeros(shp, jnp.float32)
    loy = jnp.zeros(shp, jnp.float32)
    loz = jnp.zeros(shp, jnp.float32)
    hix = jnp.ones(shp, jnp.float32)
    hiy = jnp.ones(shp, jnp.float32)
    hiz = jnp.ones(shp, jnp.float32)
    sidx = jnp.zeros(shp, jnp.int32)
    for s in range(S):
        l0 = b_ref[s, 0, 0]
        l1 = b_ref[s, 0, 1]
        l2 = b_ref[s, 0, 2]
        h0 = b_ref[s, 1, 0]
        h1 = b_ref[s, 1, 1]
        h2 = b_ref[s, 1, 2]
        m = ((px > l0) & (px < h0) & (py > l1) & (py < h1)
             & (pz > l2) & (pz < h2) & (~pre))
        pre = pre | m
        lox = jnp.where(m, l0, lox)
        loy = jnp.where(m, l1, loy)
        loz = jnp.where(m, l2, loz)
        hix = jnp.where(m, h0, hix)
        hiy = jnp.where(m, h1, hiy)
        hiz = jnp.where(m, h2, hiz)
        sidx = jnp.where(m, s, sidx)
    routed = pre.astype(jnp.float32)
    dx = jnp.where(pre, hix - lox, 1.0)
    dy = jnp.where(pre, hiy - loy, 1.0)
    dz = jnp.where(pre, hiz - loz, 1.0)
    un = (px - lox) / dx
    vn = (py - loy) / dy
    tn = (pz - loz) / dz
    sbase = sidx * (R * R)
    for o, (ca, cb) in enumerate(((un, vn), (un, tn), (vn, tn))):
        xx = jnp.clip(ca, 0.0, 1.0) * (R - 1)
        yy = jnp.clip(cb, 0.0, 1.0) * (R - 1)
        x0 = jnp.clip(jnp.floor(xx), 0, R - 2).astype(jnp.int32)
        y0 = jnp.clip(jnp.floor(yy), 0, R - 2).astype(jnp.int32)
        wx = xx - x0.astype(jnp.float32)
        wy = yy - y0.astype(jnp.float32)
        base = sbase + x0 * R + y0
        iw_ref[4 * o + 0] = base
        iw_ref[4 * o + 1] = base + 1
        iw_ref[4 * o + 2] = base + R
        iw_ref[4 * o + 3] = base + R + 1
        wq = ((1 - wx) * (1 - wy), (1 - wx) * wy,
              wx * (1 - wy), wx * wy)
        for q in range(4):
            iw_ref[12 + 4 * o + q] = lax.bitcast_convert_type(
                wq[q] * routed, jnp.int32)


def _prep(px, py, pz, boundaries):
    nb = px.shape[0]
    blk = 256
    grid = nb // blk
    return pl.pallas_call(
        _prep_body,
        grid=(grid,),
        in_specs=[
            pl.BlockSpec(memory_space=pltpu.SMEM),
            pl.BlockSpec((blk, 128), lambda i: (i, 0)),
            pl.BlockSpec((blk, 128), lambda i: (i, 0)),
            pl.BlockSpec((blk, 128), lambda i: (i, 0)),
        ],
        out_specs=pl.BlockSpec((24, blk, 128), lambda i: (0, i, 0)),
        out_shape=jax.ShapeDtypeStruct((24, nb, 128), jnp.int32),
    )(boundaries, px, py, pz)


def _sc_body(iw_hbm, t0, t1, t2, feat_hbm, *scr):
    iw_v = scr[0:2]
    rows = (scr[2:14], scr[14:26])
    outb = scr[26:28]
    gsem = scr[28:30]
    osem = scr[30:32]
    tabs = (t0, t1, t2)

    wid = lax.axis_index("s") * NC + lax.axis_index("c")
    base0 = wid * PPW
    iota16 = lax.iota(jnp.int32, 16)

    def iw_load(c, b):
        base = pl.multiple_of(base0 + c * CH, CH)
        pltpu.sync_copy(iw_hbm.at[:, pl.ds(base, CH)], iw_v[b])

    def fire(b):
        for j in range(12):
            pltpu.async_copy(tabs[j // 4].at[iw_v[b].at[j]], rows[b][j],
                             gsem[b])

    def drain_gathers(b):
        for j in range(12):
            pltpu.make_async_copy(tabs[j // 4].at[iw_v[b].at[j]],
                                  rows[b][j], gsem[b]).wait()

    def compute(b):
        for g in range(CH // 16):
            ptv = g * 16 + iota16
            wvs = [plsc.bitcast(iw_v[b][12 + j, pl.ds(g * 16, 16)],
                                jnp.float32) for j in range(12)]

            def c_body(ci, carry):
                for u in range(2):
                    cs = jnp.full((16,), u, jnp.int32) + 2 * ci
                    acc = wvs[0] * plsc.load_gather(rows[b][0], [ptv, cs])
                    for j in range(1, 12):
                        acc = acc + wvs[j] * plsc.load_gather(
                            rows[b][j], [ptv, cs])
                    plsc.store_scatter(outb[b], [ptv, cs], acc)
                return carry

            lax.fori_loop(0, D2 // 2, c_body, 0)

    def out_fire(c, b):
        base = pl.multiple_of(base0 + c * CH, CH)
        pltpu.async_copy(outb[b], feat_hbm.at[pl.ds(base, CH)], osem[b])

    def out_drain(b):
        pltpu.make_async_copy(outb[b], feat_hbm.at[pl.ds(0, CH)],
                              osem[b]).wait()

    iw_load(0, 0)
    fire(0)

    def pair_body(i, carry):
        for b in range(2):
            c = 2 * i + b
            nc = c + 1

            @pl.when(nc < NCHUNK)
            def _():
                iw_load(nc, 1 - b)
                fire(1 - b)

            drain_gathers(b)

            @pl.when(c >= 2)
            def _():
                out_drain(b)

            compute(b)
            out_fire(c, b)
        return carry

    lax.fori_loop(0, NCHUNK // 2, pair_body, 0)
    out_drain(0)
    out_drain(1)


def _gather_sc(iw, t0, t1, t2):
    mesh = plsc.VectorSubcoreMesh(
        core_axis_name="c", subcore_axis_name="s",
        num_cores=NC, num_subcores=NS)
    scratch = (
        [pltpu.VMEM((24, CH), jnp.int32) for _ in range(2)]
        + [pltpu.VMEM((CH, D2), jnp.float32) for _ in range(24)]
        + [pltpu.VMEM((CH, D2), jnp.float32) for _ in range(2)]
        + [pltpu.SemaphoreType.DMA for _ in range(4)]
    )
    fn = pl.kernel(
        _sc_body,
        out_type=jax.ShapeDtypeStruct((N, D2), jnp.float32),
        mesh=mesh,
        scratch_types=scratch,
        compiler_params=pltpu.CompilerParams(use_tc_tiling_on_sc=False,
                                             needs_layout_passes=False),
    )
    return fn(iw, t0, t1, t2)


def _mlp_body(f_ref, W0r, b0r, W1r, b1r, Wfr, bfr, out_ref):
    f = f_ref[...]
    h = jnp.maximum(jnp.dot(f, W0r[...], preferred_element_type=jnp.float32)
                    + b0r[...], 0.0)
    h = jnp.maximum(jnp.dot(h, W1r[...], preferred_element_type=jnp.float32)
                    + b1r[...], 0.0)
    z = jnp.dot(h, Wfr[...], preferred_element_type=jnp.float32) + bfr[...]
    col = lax.broadcasted_iota(jnp.int32, z.shape, 1)
    out_ref[...] = jnp.where(col < 3, jax.nn.sigmoid(z), jnp.tanh(z))


def _mlp(feat, W0c, b0c, W1c, b1c, Wf, bf):
    blk = 2048
    grid = N // blk

    def fullspec(a):
        return pl.BlockSpec(a.shape, lambda i: (0,) * a.ndim)

    ws = [W0c, b0c, W1c, b1c, Wf, bf]
    return pl.pallas_call(
        _mlp_body,
        grid=(grid,),
        in_specs=([pl.BlockSpec((blk, D2), lambda i: (i, 0))]
                  + [fullspec(a) for a in ws]),
        out_specs=pl.BlockSpec((blk, 4), lambda i: (i, 0)),
        out_shape=jax.ShapeDtypeStruct((N, 4), jnp.float32),
    )(feat, *ws)


def kernel(p, boundaries, planes_xy, planes_xz, planes_yz,
           c_planes_xy, c_planes_xz, c_planes_yz,
           W0, b0, W1, b1, Wout, bout, cW0, cb0, cW1, cb1, cWout, cbout):
    nb = N // 128
    px = p[:, 0].reshape(nb, 128)
    py = p[:, 1].reshape(nb, 128)
    pz = p[:, 2].reshape(nb, 128)
    iw3 = _prep(px, py, pz, boundaries)
    iw = iw3.reshape(24, N)

    tabs = [jnp.concatenate([a.reshape(S * R * R, IN_DIM),
                             ca.reshape(S * R * R, IN_DIM)], axis=1)
            for a, ca in ((planes_xy, c_planes_xy),
                          (planes_xz, c_planes_xz),
                          (planes_yz, c_planes_yz))]
    feat = _gather_sc(iw, *tabs)

    zz = jnp.zeros((HID, HID), jnp.float32)
    W0c = jnp.block([[W0, zz], [zz, cW0]])
    b0c = jnp.concatenate([b0, cb0]).reshape(1, D2)
    W1c = jnp.block([[W1, zz], [zz, cW1]])
    b1c = jnp.concatenate([b1, cb1]).reshape(1, D2)
    Wf = jnp.block([[jnp.zeros((HID, 3), jnp.float32), Wout],
                    [cWout, jnp.zeros((HID, 1), jnp.float32)]])
    bf = jnp.concatenate([cbout, bout]).reshape(1, 4)
    return _mlp(feat, W0c, b0c, W1c, b1c, Wf, bf)
